# baseline (speedup 1.0000x reference)
import functools

import jax
import jax.numpy as jnp
from jax import lax
from jax.experimental import pallas as pl
from jax.experimental.pallas import tpu as pltpu
from jax.experimental.pallas import tpu_sc as plsc

NC = 2
NS = 16
NW = NC * NS
LANES = 16

B = 64
ROWS = 5120
RPT = ROWS // NW
CH = 32
RPT0 = ROWS // NS
NCH0 = RPT0 // CH
HALF = 5000
HPAD = 5008
NBUF = 4
PRE = 3


def _splat_i32(v):
    return jnp.zeros((LANES,), jnp.int32) + v


def _make_deg_kernel():
    mesh = plsc.VectorSubcoreMesh(core_axis_name="c", subcore_axis_name="s")

    @functools.partial(
        pl.kernel,
        out_type=jax.ShapeDtypeStruct((NW * 2 * HPAD,), jnp.float32),
        mesh=mesh,
        compiler_params=pltpu.CompilerParams(needs_layout_passes=False),
        scratch_types=[
            pltpu.VMEM((RPT, B), jnp.int32),
            pltpu.VMEM((HPAD * LANES,), jnp.float32),
            pltpu.VMEM((HPAD,), jnp.float32),
        ],
    )
    def deg_kernel(row_hbm, out_hbm, row_buf, hist, deg_buf):
        cid = lax.axis_index("c")
        sid = lax.axis_index("s")
        wid = cid * NS + sid
        pltpu.sync_copy(row_hbm.at[pl.ds(wid * RPT, RPT)], row_buf)
        iota = lax.iota(jnp.int32, LANES)
        ones = jnp.ones((LANES,), jnp.float32)
        zeros = jnp.zeros((LANES,), jnp.float32)
        for p in range(2):
            def zero_body(i, _):
                hist[pl.ds(i * LANES, LANES)] = zeros
                return 0
            lax.fori_loop(0, HPAD, zero_body, 0)

            def acc_body(i, _):
                for s in range(B // LANES):
                    idx = row_buf[i, pl.ds(s * LANES, LANES)]
                    il = idx - p * HALF
                    m = (il >= 0) & (il < HALF)
                    ilc = jnp.minimum(jnp.maximum(il, 0), HALF - 1)
                    plsc.addupdate_scatter(
                        hist, [ilc * LANES + iota], ones, mask=m)
                return 0
            lax.fori_loop(0, RPT, acc_body, 0)

            def red_body(i, _):
                r0 = i * LANES
                acc = zeros
                ridx = (iota + r0) * LANES
                for l in range(LANES):
                    acc = acc + plsc.load_gather(hist, [ridx + l])
                deg_buf[pl.ds(r0, LANES)] = acc
                return 0
            lax.fori_loop(0, HPAD // LANES, red_body, 0)
            pltpu.sync_copy(
                deg_buf, out_hbm.at[pl.ds((wid * 2 + p) * HPAD, HPAD)])

    return deg_kernel


def _make_prop_kernel(n_nodes, d):
    split = 640
    cz = 40
    nch_main = split // cz
    nch_last = (n_nodes - split * (NS - 1)) // cz
    npad = 8
    mesh = plsc.VectorSubcoreMesh(core_axis_name="c", subcore_axis_name="s")

    @functools.partial(
        pl.kernel,
        out_type=jax.ShapeDtypeStruct((n_nodes, d), jnp.float32),
        mesh=mesh,
        compiler_params=pltpu.CompilerParams(needs_layout_passes=False),
        scratch_types=[
            pltpu.VMEM((CH, B), jnp.int32),
            pltpu.VMEM((CH, B), jnp.int32),
            pltpu.VMEM((B, d), jnp.float32),
            pltpu.VMEM((B, d), jnp.float32),
            pltpu.VMEM((B, d), jnp.float32),
            pltpu.VMEM((B, d), jnp.float32),
            pltpu.VMEM((cz, d), jnp.float32),
            pltpu.VMEM_SHARED((n_nodes + npad, d), jnp.float32),
            pltpu.SemaphoreType.DMA,
            pltpu.SemaphoreType.DMA,
            pltpu.SemaphoreType.DMA,
            pltpu.SemaphoreType.DMA,
            pltpu.SemaphoreType.DMA,
            pltpu.SemaphoreType.DMA,
            pltpu.SemaphoreType.DMA,
            pltpu.SemaphoreType.DMA,
        ],
    )
    def prop_kernel(row_hbm, col_hbm, g_hbm, out_hbm,
                    row_buf, col_buf, gb0, gb1, gb2, gb3, zbuf, acc,
                    gs0, gs1, gs2, gs3, ss0, ss1, ss2, ss3):
        gbufs = (gb0, gb1, gb2, gb3)
        gsems = (gs0, gs1, gs2, gs3)
        ssems = (ss0, ss1, ss2, ss3)
        cid = lax.axis_index("c")
        sid = lax.axis_index("s")
        wid = cid * NS + sid
        zeros = jnp.zeros((LANES,), jnp.float32)

        def zrow(i, _):
            for f in range(d // LANES):
                zbuf[i, pl.ds(f * LANES, LANES)] = zeros
            return 0
        lax.fori_loop(0, cz, zrow, 0)
        start = sid * split
        nch = jnp.where(sid < NS - 1, nch_main, nch_last)
        nch = jnp.where(cid == 0, nch, 0)

        def zcopy(t, _):
            pltpu.sync_copy(zbuf, acc.at[pl.ds(start + t * cz, cz)])
            return 0
        lax.fori_loop(0, nch, zcopy, 0)
        plsc.subcore_barrier()

        nchunk = jnp.where(cid == 0, NCH0, 0)

        def chunk_body(c, _):
            base = sid * RPT0 + c * CH
            pltpu.sync_copy(row_hbm.at[pl.ds(base, CH)], row_buf)
            pltpu.sync_copy(col_hbm.at[pl.ds(base, CH)], col_buf)
            gd = [None] * CH
            sd = [None] * CH
            for b in range(PRE):
                gd[b] = pltpu.async_copy(
                    g_hbm.at[col_buf.at[b]], gbufs[b % NBUF], gsems[b % NBUF])
            for b in range(CH):
                if b + PRE < CH:
                    if b - 1 >= 0:
                        sd[b - 1].wait()
                    gd[b + PRE] = pltpu.async_copy(
                        g_hbm.at[col_buf.at[b + PRE]],
                        gbufs[(b + PRE) % NBUF], gsems[(b + PRE) % NBUF])
                gd[b].wait()
                sd[b] = pltpu.async_copy(
                    gbufs[b % NBUF], acc.at[row_buf.at[b]], ssems[b % NBUF],
                    add=True)
            for b in range(CH - PRE - 1, CH):
                if b >= 0:
                    sd[b].wait()
            return 0
        lax.fori_loop(0, nchunk, chunk_body, 0)
        plsc.subcore_barrier()

        def wcopy(t, _):
            r0 = start + t * cz
            pltpu.sync_copy(acc.at[pl.ds(r0, cz)], zbuf)
            pltpu.sync_copy(zbuf, out_hbm.at[pl.ds(r0, cz)])
            return 0
        lax.fori_loop(0, nch, wcopy, 0)

    return prop_kernel


def _dinv_body(parts_ref, out_ref):
    deg = jnp.sum(parts_ref[...], axis=0, keepdims=True)
    out_ref[...] = jnp.where(deg > 0, lax.rsqrt(deg), 0.0)


def _prescale_body(x_ref, dv_ref, g_ref):
    g_ref[...] = dv_ref[...] * x_ref[...]


def _combine_first_body(x_ref, p_ref, dv_ref, w0_ref, w1_ref, b_ref,
                        t1_ref, g1_ref, out_ref):
    dv = dv_ref[...]
    t1 = -dv * p_ref[...]
    t1_ref[...] = t1
    g1_ref[...] = dv * t1
    out_ref[...] = (
        jnp.dot(x_ref[...], w0_ref[...], preferred_element_type=jnp.float32)
        + jnp.dot(t1, w1_ref[...], preferred_element_type=jnp.float32)
        + b_ref[...])


def _combine_step_body(p_ref, dv_ref, tx0_ref, out_in_ref, w_ref,
                       t2_ref, g2_ref, out_ref):
    dv = dv_ref[...]
    t2 = -2.0 * dv * p_ref[...] - tx0_ref[...]
    t2_ref[...] = t2
    g2_ref[...] = dv * t2
    out_ref[...] = out_in_ref[...] + jnp.dot(
        t2, w_ref[...], preferred_element_type=jnp.float32)


def kernel(x, edge_index, W, b):
    n, d = x.shape
    e = edge_index.shape[1]
    k_hops = W.shape[0]
    assert e % B == 0 and e <= ROWS * B and n % 8 == 0
    pad = ROWS * B - e

    rowp = jnp.concatenate(
        [edge_index[0],
         n + (jnp.arange(pad, dtype=jnp.int32) % 8)]).reshape(ROWS, B)
    colp = jnp.concatenate(
        [edge_index[1], jnp.zeros((pad,), jnp.int32)]).reshape(ROWS, B)

    deg_parts = _make_deg_kernel()(rowp)
    deg_parts = deg_parts.reshape(NW * 2, HPAD)[:, :HALF].reshape(NW, n)

    dinv = pl.pallas_call(
        _dinv_body,
        out_shape=jax.ShapeDtypeStruct((1, n), jnp.float32),
    )(deg_parts)
    dv2d = dinv.reshape(n, 1)

    prop = _make_prop_kernel(n, d)

    rb = 1000
    grid = (n // rb,)
    blk = pl.BlockSpec((rb, d), lambda i: (i, 0))
    dvblk = pl.BlockSpec((rb, 1), lambda i: (i, 0))
    wblk = pl.BlockSpec((d, d), lambda i: (0, 0))
    bblk = pl.BlockSpec((1, d), lambda i: (0, 0))
    nd = jax.ShapeDtypeStruct((n, d), jnp.float32)

    prescale = pl.pallas_call(
        _prescale_body,
        grid=grid,
        in_specs=[blk, dvblk],
        out_specs=blk,
        out_shape=nd,
    )
    combine_first = pl.pallas_call(
        _combine_first_body,
        grid=grid,
        in_specs=[blk, blk, dvblk, wblk, wblk, bblk],
        out_specs=(blk, blk, blk),
        out_shape=(nd, nd, nd),
    )
    combine_step = pl.pallas_call(
        _combine_step_body,
        grid=grid,
        in_specs=[blk, dvblk, blk, blk, wblk],
        out_specs=(blk, blk, blk),
        out_shape=(nd, nd, nd),
    )

    g = prescale(x, dv2d)
    parts = prop(rowp, colp, g)
    tx1, g1, out = combine_first(x, parts, dv2d, W[0], W[1],
                                 b.reshape(1, d))
    tx0, gcur = x, g1
    for k in range(2, k_hops):
        parts = prop(rowp, colp, gcur)
        tx2, g2, out = combine_step(parts, dv2d, tx0, out, W[k])
        tx0, tx1, gcur = tx1, tx2, g2
    return out


# speedup vs baseline: 1.0002x; 1.0002x over previous
import functools

import jax
import jax.numpy as jnp
from jax import lax
from jax.experimental import pallas as pl
from jax.experimental.pallas import tpu as pltpu
from jax.experimental.pallas import tpu_sc as plsc

NC = 2
NS = 16
NW = NC * NS
LANES = 16

B = 64
ROWS = 5120
RPT = ROWS // NW
CH = 32
RPT0 = ROWS // NS
NCH0 = RPT0 // CH
HALF = 5000
HPAD = 5008
NBUF = 4
PRE = 3


def _splat_i32(v):
    return jnp.zeros((LANES,), jnp.int32) + v


def _make_deg_kernel():
    mesh = plsc.VectorSubcoreMesh(core_axis_name="c", subcore_axis_name="s")

    @functools.partial(
        pl.kernel,
        out_type=jax.ShapeDtypeStruct((NW * 2 * HPAD,), jnp.float32),
        mesh=mesh,
        compiler_params=pltpu.CompilerParams(needs_layout_passes=False),
        scratch_types=[
            pltpu.VMEM((RPT, B), jnp.int32),
            pltpu.VMEM((HPAD * LANES,), jnp.float32),
            pltpu.VMEM((HPAD,), jnp.float32),
        ],
    )
    def deg_kernel(row_hbm, out_hbm, row_buf, hist, deg_buf):
        cid = lax.axis_index("c")
        sid = lax.axis_index("s")
        wid = cid * NS + sid
        pltpu.sync_copy(row_hbm.at[pl.ds(wid * RPT, RPT)], row_buf)
        iota = lax.iota(jnp.int32, LANES)
        ones = jnp.ones((LANES,), jnp.float32)
        zeros = jnp.zeros((LANES,), jnp.float32)
        for p in range(2):
            def zero_body(i, _):
                hist[pl.ds(i * LANES, LANES)] = zeros
                return 0
            lax.fori_loop(0, HPAD, zero_body, 0)

            def acc_body(i, _):
                for s in range(B // LANES):
                    idx = row_buf[i, pl.ds(s * LANES, LANES)]
                    il = idx - p * HALF
                    m = (il >= 0) & (il < HALF)
                    ilc = jnp.minimum(jnp.maximum(il, 0), HALF - 1)
                    plsc.addupdate_scatter(
                        hist, [ilc * LANES + iota], ones, mask=m)
                return 0
            lax.fori_loop(0, RPT, acc_body, 0)

            def red_body(i, _):
                r0 = i * LANES
                acc = zeros
                ridx = (iota + r0) * LANES
                for l in range(LANES):
                    acc = acc + plsc.load_gather(hist, [ridx + l])
                deg_buf[pl.ds(r0, LANES)] = acc
                return 0
            lax.fori_loop(0, HPAD // LANES, red_body, 0)
            pltpu.sync_copy(
                deg_buf, out_hbm.at[pl.ds((wid * 2 + p) * HPAD, HPAD)])

    return deg_kernel


def _make_prop_kernel(n_nodes, d):
    split = 640
    cz = 40
    nch_main = split // cz
    nch_last = (n_nodes - split * (NS - 1)) // cz
    npad = 128
    mesh = plsc.VectorSubcoreMesh(core_axis_name="c", subcore_axis_name="s")

    @functools.partial(
        pl.kernel,
        out_type=jax.ShapeDtypeStruct((n_nodes, d), jnp.float32),
        mesh=mesh,
        compiler_params=pltpu.CompilerParams(needs_layout_passes=False),
        scratch_types=[
            pltpu.VMEM((CH, B), jnp.int32),
            pltpu.VMEM((CH, B), jnp.int32),
            pltpu.VMEM((B, d), jnp.float32),
            pltpu.VMEM((B, d), jnp.float32),
            pltpu.VMEM((B, d), jnp.float32),
            pltpu.VMEM((B, d), jnp.float32),
            pltpu.VMEM((cz, d), jnp.float32),
            pltpu.VMEM_SHARED((n_nodes + npad, d), jnp.float32),
            pltpu.SemaphoreType.DMA,
            pltpu.SemaphoreType.DMA,
            pltpu.SemaphoreType.DMA,
            pltpu.SemaphoreType.DMA,
            pltpu.SemaphoreType.DMA,
            pltpu.SemaphoreType.DMA,
            pltpu.SemaphoreType.DMA,
            pltpu.SemaphoreType.DMA,
        ],
    )
    def prop_kernel(row_hbm, col_hbm, g_hbm, out_hbm,
                    row_buf, col_buf, gb0, gb1, gb2, gb3, zbuf, acc,
                    gs0, gs1, gs2, gs3, ss0, ss1, ss2, ss3):
        gbufs = (gb0, gb1, gb2, gb3)
        gsems = (gs0, gs1, gs2, gs3)
        ssems = (ss0, ss1, ss2, ss3)
        cid = lax.axis_index("c")
        sid = lax.axis_index("s")
        wid = cid * NS + sid
        zeros = jnp.zeros((LANES,), jnp.float32)

        def zrow(i, _):
            for f in range(d // LANES):
                zbuf[i, pl.ds(f * LANES, LANES)] = zeros
            return 0
        lax.fori_loop(0, cz, zrow, 0)
        start = sid * split
        nch = jnp.where(sid < NS - 1, nch_main, nch_last)
        nch = jnp.where(cid == 0, nch, 0)

        def zcopy(t, _):
            pltpu.sync_copy(zbuf, acc.at[pl.ds(start + t * cz, cz)])
            return 0
        lax.fori_loop(0, nch, zcopy, 0)
        plsc.subcore_barrier()

        nchunk = jnp.where(cid == 0, NCH0, 0)

        def chunk_body(c, _):
            base = sid * RPT0 + c * CH
            pltpu.sync_copy(row_hbm.at[pl.ds(base, CH)], row_buf)
            pltpu.sync_copy(col_hbm.at[pl.ds(base, CH)], col_buf)
            gd = [None] * CH
            sd = [None] * CH
            for b in range(PRE):
                gd[b] = pltpu.async_copy(
                    g_hbm.at[col_buf.at[b]], gbufs[b % NBUF], gsems[b % NBUF])
            for b in range(CH):
                if b + PRE < CH:
                    if b - 1 >= 0:
                        sd[b - 1].wait()
                    gd[b + PRE] = pltpu.async_copy(
                        g_hbm.at[col_buf.at[b + PRE]],
                        gbufs[(b + PRE) % NBUF], gsems[(b + PRE) % NBUF])
                gd[b].wait()
                sd[b] = pltpu.async_copy(
                    gbufs[b % NBUF], acc.at[row_buf.at[b]], ssems[b % NBUF],
                    add=True)
            for b in range(CH - PRE - 1, CH):
                if b >= 0:
                    sd[b].wait()
            return 0
        lax.fori_loop(0, nchunk, chunk_body, 0)
        plsc.subcore_barrier()

        def wcopy(t, _):
            r0 = start + t * cz
            pltpu.sync_copy(acc.at[pl.ds(r0, cz)], zbuf)
            pltpu.sync_copy(zbuf, out_hbm.at[pl.ds(r0, cz)])
            return 0
        lax.fori_loop(0, nch, wcopy, 0)

    return prop_kernel


def _dinv_body(parts_ref, out_ref):
    deg = jnp.sum(parts_ref[...], axis=0, keepdims=True)
    out_ref[...] = jnp.where(deg > 0, lax.rsqrt(deg), 0.0)


def _prescale_body(x_ref, dv_ref, g_ref):
    g_ref[...] = dv_ref[...] * x_ref[...]


def _combine_first_body(x_ref, p_ref, dv_ref, w0_ref, w1_ref, b_ref,
                        t1_ref, g1_ref, out_ref):
    dv = dv_ref[...]
    t1 = -dv * p_ref[...]
    t1_ref[...] = t1
    g1_ref[...] = dv * t1
    out_ref[...] = (
        jnp.dot(x_ref[...], w0_ref[...], preferred_element_type=jnp.float32)
        + jnp.dot(t1, w1_ref[...], preferred_element_type=jnp.float32)
        + b_ref[...])


def _combine_step_body(p_ref, dv_ref, tx0_ref, out_in_ref, w_ref,
                       t2_ref, g2_ref, out_ref):
    dv = dv_ref[...]
    t2 = -2.0 * dv * p_ref[...] - tx0_ref[...]
    t2_ref[...] = t2
    g2_ref[...] = dv * t2
    out_ref[...] = out_in_ref[...] + jnp.dot(
        t2, w_ref[...], preferred_element_type=jnp.float32)


def kernel(x, edge_index, W, b):
    n, d = x.shape
    e = edge_index.shape[1]
    k_hops = W.shape[0]
    assert e % B == 0 and e <= ROWS * B and n % 8 == 0
    pad = ROWS * B - e

    rowp = jnp.concatenate(
        [edge_index[0],
         n + (jnp.arange(pad, dtype=jnp.int32) % 128)]).reshape(ROWS, B)
    colp = jnp.concatenate(
        [edge_index[1], jnp.zeros((pad,), jnp.int32)]).reshape(ROWS, B)

    deg_parts = _make_deg_kernel()(rowp)
    deg_parts = deg_parts.reshape(NW * 2, HPAD)[:, :HALF].reshape(NW, n)

    dinv = pl.pallas_call(
        _dinv_body,
        out_shape=jax.ShapeDtypeStruct((1, n), jnp.float32),
    )(deg_parts)
    dv2d = dinv.reshape(n, 1)

    prop = _make_prop_kernel(n, d)

    rb = 1000
    grid = (n // rb,)
    blk = pl.BlockSpec((rb, d), lambda i: (i, 0))
    dvblk = pl.BlockSpec((rb, 1), lambda i: (i, 0))
    wblk = pl.BlockSpec((d, d), lambda i: (0, 0))
    bblk = pl.BlockSpec((1, d), lambda i: (0, 0))
    nd = jax.ShapeDtypeStruct((n, d), jnp.float32)

    prescale = pl.pallas_call(
        _prescale_body,
        grid=grid,
        in_specs=[blk, dvblk],
        out_specs=blk,
        out_shape=nd,
    )
    combine_first = pl.pallas_call(
        _combine_first_body,
        grid=grid,
        in_specs=[blk, blk, dvblk, wblk, wblk, bblk],
        out_specs=(blk, blk, blk),
        out_shape=(nd, nd, nd),
    )
    combine_step = pl.pallas_call(
        _combine_step_body,
        grid=grid,
        in_specs=[blk, dvblk, blk, blk, wblk],
        out_specs=(blk, blk, blk),
        out_shape=(nd, nd, nd),
    )

    g = prescale(x, dv2d)
    parts = prop(rowp, colp, g)
    tx1, g1, out = combine_first(x, parts, dv2d, W[0], W[1],
                                 b.reshape(1, d))
    tx0, gcur = x, g1
    for k in range(2, k_hops):
        parts = prop(rowp, colp, gcur)
        tx2, g2, out = combine_step(parts, dv2d, tx0, out, W[k])
        tx0, tx1, gcur = tx1, tx2, g2
    return out


# speedup vs baseline: 1.0264x; 1.0262x over previous
import functools

import jax
import jax.numpy as jnp
from jax import lax
from jax.experimental import pallas as pl
from jax.experimental.pallas import tpu as pltpu
from jax.experimental.pallas import tpu_sc as plsc

NC = 2
NS = 16
NW = NC * NS
LANES = 16

B = 64
ROWS = 5120
RPT = ROWS // NW
CH = 32
RPT0 = ROWS // NS
NCH0 = RPT0 // CH
HALF = 5000
HPAD = 5008
NBUF = 4
PRE = 3


def _splat_i32(v):
    return jnp.zeros((LANES,), jnp.int32) + v


def _make_deg_kernel():
    mesh = plsc.VectorSubcoreMesh(core_axis_name="c", subcore_axis_name="s")

    @functools.partial(
        pl.kernel,
        out_type=jax.ShapeDtypeStruct((NW * 2 * HPAD,), jnp.float32),
        mesh=mesh,
        compiler_params=pltpu.CompilerParams(needs_layout_passes=False),
        scratch_types=[
            pltpu.VMEM((RPT, B), jnp.int32),
            pltpu.VMEM((HPAD * LANES,), jnp.float32),
            pltpu.VMEM((HPAD,), jnp.float32),
        ],
    )
    def deg_kernel(row_hbm, out_hbm, row_buf, hist, deg_buf):
        cid = lax.axis_index("c")
        sid = lax.axis_index("s")
        wid = cid * NS + sid
        pltpu.sync_copy(row_hbm.at[pl.ds(wid * RPT, RPT)], row_buf)
        iota = lax.iota(jnp.int32, LANES)
        ones = jnp.ones((LANES,), jnp.float32)
        zeros = jnp.zeros((LANES,), jnp.float32)
        for p in range(2):
            def zero_body(i, _):
                hist[pl.ds(i * LANES, LANES)] = zeros
                return 0
            lax.fori_loop(0, HPAD, zero_body, 0)

            def acc_body(i, _):
                for s in range(B // LANES):
                    idx = row_buf[i, pl.ds(s * LANES, LANES)]
                    il = idx - p * HALF
                    m = (il >= 0) & (il < HALF)
                    ilc = jnp.minimum(jnp.maximum(il, 0), HALF - 1)
                    plsc.addupdate_scatter(
                        hist, [ilc * LANES + iota], ones, mask=m)
                return 0
            lax.fori_loop(0, RPT, acc_body, 0)

            def red_body(i, _):
                r0 = i * LANES
                acc = zeros
                ridx = (iota + r0) * LANES
                for l in range(LANES):
                    acc = acc + plsc.load_gather(hist, [ridx + l])
                deg_buf[pl.ds(r0, LANES)] = acc
                return 0
            lax.fori_loop(0, HPAD // LANES, red_body, 0)
            pltpu.sync_copy(
                deg_buf, out_hbm.at[pl.ds((wid * 2 + p) * HPAD, HPAD)])

    return deg_kernel


def _make_prop_kernel(n_nodes, d):
    split = 640
    cz = 40
    nch_main = split // cz
    nch_last = (n_nodes - split * (NS - 1)) // cz
    npad = 128
    mesh = plsc.VectorSubcoreMesh(core_axis_name="c", subcore_axis_name="s")

    @functools.partial(
        pl.kernel,
        out_type=jax.ShapeDtypeStruct((n_nodes, d), jnp.float32),
        mesh=mesh,
        compiler_params=pltpu.CompilerParams(needs_layout_passes=False),
        scratch_types=[
            pltpu.VMEM((CH, B), jnp.int32),
            pltpu.VMEM((CH, B), jnp.int32),
            pltpu.VMEM((B, d), jnp.float32),
            pltpu.VMEM((B, d), jnp.float32),
            pltpu.VMEM((B, d), jnp.float32),
            pltpu.VMEM((B, d), jnp.float32),
            pltpu.VMEM((cz, d), jnp.float32),
            pltpu.VMEM_SHARED((n_nodes + npad, d), jnp.float32),
            pltpu.SemaphoreType.DMA,
            pltpu.SemaphoreType.DMA,
            pltpu.SemaphoreType.DMA,
            pltpu.SemaphoreType.DMA,
            pltpu.SemaphoreType.DMA,
            pltpu.SemaphoreType.DMA,
            pltpu.SemaphoreType.DMA,
            pltpu.SemaphoreType.DMA,
        ],
    )
    def prop_kernel(row_hbm, col_hbm, g_hbm, out_hbm,
                    row_buf, col_buf, gb0, gb1, gb2, gb3, zbuf, acc,
                    gs0, gs1, gs2, gs3, ss0, ss1, ss2, ss3):
        gbufs = (gb0, gb1, gb2, gb3)
        gsems = (gs0, gs1, gs2, gs3)
        ssems = (ss0, ss1, ss2, ss3)
        cid = lax.axis_index("c")
        sid = lax.axis_index("s")
        wid = cid * NS + sid
        zeros = jnp.zeros((LANES,), jnp.float32)

        def zrow(i, _):
            for f in range(d // LANES):
                zbuf[i, pl.ds(f * LANES, LANES)] = zeros
            return 0
        lax.fori_loop(0, cz, zrow, 0)
        start = sid * split
        nch = jnp.where(sid < NS - 1, nch_main, nch_last)
        nch = jnp.where(cid == 0, nch, 0)

        def zcopy(t, _):
            pltpu.sync_copy(zbuf, acc.at[pl.ds(start + t * cz, cz)])
            return 0
        lax.fori_loop(0, nch, zcopy, 0)
        plsc.subcore_barrier()

        nchunk = jnp.where(cid == 0, NCH0, 0)

        def chunk_body(c, _):
            base = sid * RPT0 + c * CH
            pltpu.sync_copy(row_hbm.at[pl.ds(base, CH)], row_buf)
            pltpu.sync_copy(col_hbm.at[pl.ds(base, CH)], col_buf)
            gd = [None] * CH
            sd = [None] * CH
            for b in range(PRE):
                gd[b] = pltpu.async_copy(
                    g_hbm.at[col_buf.at[b]], gbufs[b % NBUF], gsems[b % NBUF])
            for b in range(CH):
                if b + PRE < CH:
                    if b - 1 >= 0:
                        pass
                    gd[b + PRE] = pltpu.async_copy(
                        g_hbm.at[col_buf.at[b + PRE]],
                        gbufs[(b + PRE) % NBUF], gsems[(b + PRE) % NBUF])
                gd[b].wait()
                sd[b] = None
            for b in range(CH - PRE - 1, CH):
                if b >= 0:
                    pass
            return 0
        lax.fori_loop(0, nchunk, chunk_body, 0)
        plsc.subcore_barrier()

        def wcopy(t, _):
            r0 = start + t * cz
            pltpu.sync_copy(acc.at[pl.ds(r0, cz)], zbuf)
            pltpu.sync_copy(zbuf, out_hbm.at[pl.ds(r0, cz)])
            return 0
        lax.fori_loop(0, nch, wcopy, 0)

    return prop_kernel


def _dinv_body(parts_ref, out_ref):
    deg = jnp.sum(parts_ref[...], axis=0, keepdims=True)
    out_ref[...] = jnp.where(deg > 0, lax.rsqrt(deg), 0.0)


def _prescale_body(x_ref, dv_ref, g_ref):
    g_ref[...] = dv_ref[...] * x_ref[...]


def _combine_first_body(x_ref, p_ref, dv_ref, w0_ref, w1_ref, b_ref,
                        t1_ref, g1_ref, out_ref):
    dv = dv_ref[...]
    t1 = -dv * p_ref[...]
    t1_ref[...] = t1
    g1_ref[...] = dv * t1
    out_ref[...] = (
        jnp.dot(x_ref[...], w0_ref[...], preferred_element_type=jnp.float32)
        + jnp.dot(t1, w1_ref[...], preferred_element_type=jnp.float32)
        + b_ref[...])


def _combine_step_body(p_ref, dv_ref, tx0_ref, out_in_ref, w_ref,
                       t2_ref, g2_ref, out_ref):
    dv = dv_ref[...]
    t2 = -2.0 * dv * p_ref[...] - tx0_ref[...]
    t2_ref[...] = t2
    g2_ref[...] = dv * t2
    out_ref[...] = out_in_ref[...] + jnp.dot(
        t2, w_ref[...], preferred_element_type=jnp.float32)


def kernel(x, edge_index, W, b):
    n, d = x.shape
    e = edge_index.shape[1]
    k_hops = W.shape[0]
    assert e % B == 0 and e <= ROWS * B and n % 8 == 0
    pad = ROWS * B - e

    rowp = jnp.concatenate(
        [edge_index[0],
         n + (jnp.arange(pad, dtype=jnp.int32) % 128)]).reshape(ROWS, B)
    colp = jnp.concatenate(
        [edge_index[1], jnp.zeros((pad,), jnp.int32)]).reshape(ROWS, B)

    deg_parts = _make_deg_kernel()(rowp)
    deg_parts = deg_parts.reshape(NW * 2, HPAD)[:, :HALF].reshape(NW, n)

    dinv = pl.pallas_call(
        _dinv_body,
        out_shape=jax.ShapeDtypeStruct((1, n), jnp.float32),
    )(deg_parts)
    dv2d = dinv.reshape(n, 1)

    prop = _make_prop_kernel(n, d)

    rb = 1000
    grid = (n // rb,)
    blk = pl.BlockSpec((rb, d), lambda i: (i, 0))
    dvblk = pl.BlockSpec((rb, 1), lambda i: (i, 0))
    wblk = pl.BlockSpec((d, d), lambda i: (0, 0))
    bblk = pl.BlockSpec((1, d), lambda i: (0, 0))
    nd = jax.ShapeDtypeStruct((n, d), jnp.float32)

    prescale = pl.pallas_call(
        _prescale_body,
        grid=grid,
        in_specs=[blk, dvblk],
        out_specs=blk,
        out_shape=nd,
    )
    combine_first = pl.pallas_call(
        _combine_first_body,
        grid=grid,
        in_specs=[blk, blk, dvblk, wblk, wblk, bblk],
        out_specs=(blk, blk, blk),
        out_shape=(nd, nd, nd),
    )
    combine_step = pl.pallas_call(
        _combine_step_body,
        grid=grid,
        in_specs=[blk, dvblk, blk, blk, wblk],
        out_specs=(blk, blk, blk),
        out_shape=(nd, nd, nd),
    )

    g = prescale(x, dv2d)
    parts = prop(rowp, colp, g)
    tx1, g1, out = combine_first(x, parts, dv2d, W[0], W[1],
                                 b.reshape(1, d))
    tx0, gcur = x, g1
    for k in range(2, k_hops):
        parts = prop(rowp, colp, gcur)
        tx2, g2, out = combine_step(parts, dv2d, tx0, out, W[k])
        tx0, tx1, gcur = tx1, tx2, g2
    return out


# speedup vs baseline: 2.4024x; 2.3406x over previous
import functools

import jax
import jax.numpy as jnp
from jax import lax
from jax.experimental import pallas as pl
from jax.experimental.pallas import tpu as pltpu
from jax.experimental.pallas import tpu_sc as plsc

NC = 2
NS = 16
NW = NC * NS
LANES = 16

B = 64
ROWS = 5120
RPT = ROWS // NW
CH = 32
RPT0 = ROWS // NS
NCH0 = RPT0 // CH
HALF = 5000
HPAD = 5008
NBUF = 4
PRE = 3


def _splat_i32(v):
    return jnp.zeros((LANES,), jnp.int32) + v


def _make_deg_kernel():
    mesh = plsc.VectorSubcoreMesh(core_axis_name="c", subcore_axis_name="s")

    @functools.partial(
        pl.kernel,
        out_type=jax.ShapeDtypeStruct((NW * 2 * HPAD,), jnp.float32),
        mesh=mesh,
        compiler_params=pltpu.CompilerParams(needs_layout_passes=False),
        scratch_types=[
            pltpu.VMEM((RPT, B), jnp.int32),
            pltpu.VMEM((HPAD * LANES,), jnp.float32),
            pltpu.VMEM((HPAD,), jnp.float32),
        ],
    )
    def deg_kernel(row_hbm, out_hbm, row_buf, hist, deg_buf):
        cid = lax.axis_index("c")
        sid = lax.axis_index("s")
        wid = cid * NS + sid
        pltpu.sync_copy(row_hbm.at[pl.ds(wid * RPT, RPT)], row_buf)
        iota = lax.iota(jnp.int32, LANES)
        ones = jnp.ones((LANES,), jnp.float32)
        zeros = jnp.zeros((LANES,), jnp.float32)
        for p in range(2):
            def zero_body(i, _):
                hist[pl.ds(i * LANES, LANES)] = zeros
                return 0
            lax.fori_loop(0, HPAD, zero_body, 0)

            def acc_body(i, _):
                for s in range(B // LANES):
                    idx = row_buf[i, pl.ds(s * LANES, LANES)]
                    il = idx - p * HALF
                    m = (il >= 0) & (il < HALF)
                    ilc = jnp.minimum(jnp.maximum(il, 0), HALF - 1)
                    plsc.addupdate_scatter(
                        hist, [ilc * LANES + iota], ones, mask=m)
                return 0
            lax.fori_loop(0, RPT, acc_body, 0)

            def red_body(i, _):
                r0 = i * LANES
                acc = zeros
                ridx = (iota + r0) * LANES
                for l in range(LANES):
                    acc = acc + plsc.load_gather(hist, [ridx + l])
                deg_buf[pl.ds(r0, LANES)] = acc
                return 0
            lax.fori_loop(0, HPAD // LANES, red_body, 0)
            pltpu.sync_copy(
                deg_buf, out_hbm.at[pl.ds((wid * 2 + p) * HPAD, HPAD)])

    return deg_kernel


def _make_prop_kernel(n_nodes, d):
    split = 640
    cz = 40
    nch_main = split // cz
    nch_last = (n_nodes - split * (NS - 1)) // cz
    npad = 8
    mesh = plsc.VectorSubcoreMesh(core_axis_name="c", subcore_axis_name="s")

    @functools.partial(
        pl.kernel,
        out_type=jax.ShapeDtypeStruct((n_nodes, d), jnp.float32),
        mesh=mesh,
        compiler_params=pltpu.CompilerParams(needs_layout_passes=False),
        scratch_types=[
            pltpu.VMEM((CH, B), jnp.int32),
            pltpu.VMEM((CH, B), jnp.int32),
            pltpu.VMEM((B, d), jnp.float32),
            pltpu.VMEM((B, d), jnp.float32),
            pltpu.VMEM((B, d), jnp.float32),
            pltpu.VMEM((B, d), jnp.float32),
            pltpu.VMEM((cz, d), jnp.float32),
            pltpu.VMEM_SHARED((n_nodes + npad, d), jnp.float32),
            pltpu.SemaphoreType.DMA,
            pltpu.SemaphoreType.DMA,
            pltpu.SemaphoreType.DMA,
            pltpu.SemaphoreType.DMA,
            pltpu.SemaphoreType.DMA,
            pltpu.SemaphoreType.DMA,
            pltpu.SemaphoreType.DMA,
            pltpu.SemaphoreType.DMA,
        ],
    )
    def prop_kernel(row_hbm, col_hbm, g_hbm, out_hbm,
                    row_buf, col_buf, gb0, gb1, gb2, gb3, zbuf, acc,
                    gs0, gs1, gs2, gs3, ss0, ss1, ss2, ss3):
        gbufs = (gb0, gb1, gb2, gb3)
        gsems = (gs0, gs1, gs2, gs3)
        ssems = (ss0, ss1, ss2, ss3)
        cid = lax.axis_index("c")
        sid = lax.axis_index("s")
        wid = cid * NS + sid
        zeros = jnp.zeros((LANES,), jnp.float32)

        def zrow(i, _):
            for f in range(d // LANES):
                zbuf[i, pl.ds(f * LANES, LANES)] = zeros
            return 0
        lax.fori_loop(0, cz, zrow, 0)
        start = sid * split
        nch = jnp.where(sid < NS - 1, nch_main, nch_last)
        nch = jnp.where(cid == 0, nch, 0)

        def zcopy(t, _):
            pltpu.sync_copy(zbuf, acc.at[pl.ds(start + t * cz, cz)])
            return 0
        lax.fori_loop(0, nch, zcopy, 0)
        plsc.subcore_barrier()

        nchunk = jnp.where(cid == 0, NCH0, 0)

        def chunk_body(c, _):
            base = sid * RPT0 + c * CH
            pltpu.sync_copy(row_hbm.at[pl.ds(base, CH)], row_buf)
            pltpu.sync_copy(col_hbm.at[pl.ds(base, CH)], col_buf)
            gd = [None] * CH
            sd = [None] * CH
            for b in range(PRE):
                gd[b] = pltpu.async_copy(
                    g_hbm.at[col_buf.at[b]], gbufs[b % NBUF], gsems[b % NBUF])
            for b in range(CH):
                if b + PRE < CH:
                    if b - 1 >= 0:
                        sd[b - 1].wait()
                    gd[b + PRE] = pltpu.async_copy(
                        g_hbm.at[col_buf.at[b + PRE]],
                        gbufs[(b + PRE) % NBUF], gsems[(b + PRE) % NBUF])
                gd[b].wait()
                sd[b] = pltpu.async_copy(
                    gbufs[b % NBUF], acc.at[row_buf.at[b]], ssems[b % NBUF],
                    add=True)
            for b in range(CH - PRE - 1, CH):
                if b >= 0:
                    sd[b].wait()
            return 0
        lax.fori_loop(0, nchunk, chunk_body, 0)
        plsc.subcore_barrier()

        def wcopy(t, _):
            r0 = start + t * cz
            pltpu.sync_copy(acc.at[pl.ds(r0, cz)], zbuf)
            pltpu.sync_copy(zbuf, out_hbm.at[pl.ds(r0, cz)])
            return 0
        lax.fori_loop(0, nch, wcopy, 0)

    return prop_kernel


def _dinv_body(parts_ref, out_ref):
    deg = jnp.sum(parts_ref[...], axis=0, keepdims=True)
    out_ref[...] = jnp.where(deg > 0, lax.rsqrt(deg), 0.0)


def _prescale_body(x_ref, dv_ref, g_ref):
    g_ref[...] = dv_ref[...] * x_ref[...]


def _combine_first_body(x_ref, p_ref, dv_ref, w0_ref, w1_ref, b_ref,
                        t1_ref, g1_ref, out_ref):
    dv = dv_ref[...]
    t1 = -dv * p_ref[...]
    t1_ref[...] = t1
    g1_ref[...] = dv * t1
    out_ref[...] = (
        jnp.dot(x_ref[...], w0_ref[...], preferred_element_type=jnp.float32)
        + jnp.dot(t1, w1_ref[...], preferred_element_type=jnp.float32)
        + b_ref[...])


def _combine_step_body(p_ref, dv_ref, tx0_ref, out_in_ref, w_ref,
                       t2_ref, g2_ref, out_ref):
    dv = dv_ref[...]
    t2 = -2.0 * dv * p_ref[...] - tx0_ref[...]
    t2_ref[...] = t2
    g2_ref[...] = dv * t2
    out_ref[...] = out_in_ref[...] + jnp.dot(
        t2, w_ref[...], preferred_element_type=jnp.float32)


def kernel(x, edge_index, W, b):
    n, d = x.shape
    e = edge_index.shape[1]
    k_hops = W.shape[0]
    assert e % B == 0 and e <= ROWS * B and n % 8 == 0
    pad = ROWS * B - e

    rowp = jnp.concatenate(
        [edge_index[0],
         n + (jnp.arange(pad, dtype=jnp.int32) % 8)]).reshape(ROWS, B)
    colp = jnp.concatenate(
        [edge_index[1], jnp.arange(pad, dtype=jnp.int32) % n]).reshape(ROWS, B)

    deg_parts = _make_deg_kernel()(rowp)
    deg_parts = deg_parts.reshape(NW * 2, HPAD)[:, :HALF].reshape(NW, n)

    dinv = pl.pallas_call(
        _dinv_body,
        out_shape=jax.ShapeDtypeStruct((1, n), jnp.float32),
    )(deg_parts)
    dv2d = dinv.reshape(n, 1)

    prop = _make_prop_kernel(n, d)

    rb = 1000
    grid = (n // rb,)
    blk = pl.BlockSpec((rb, d), lambda i: (i, 0))
    dvblk = pl.BlockSpec((rb, 1), lambda i: (i, 0))
    wblk = pl.BlockSpec((d, d), lambda i: (0, 0))
    bblk = pl.BlockSpec((1, d), lambda i: (0, 0))
    nd = jax.ShapeDtypeStruct((n, d), jnp.float32)

    prescale = pl.pallas_call(
        _prescale_body,
        grid=grid,
        in_specs=[blk, dvblk],
        out_specs=blk,
        out_shape=nd,
    )
    combine_first = pl.pallas_call(
        _combine_first_body,
        grid=grid,
        in_specs=[blk, blk, dvblk, wblk, wblk, bblk],
        out_specs=(blk, blk, blk),
        out_shape=(nd, nd, nd),
    )
    combine_step = pl.pallas_call(
        _combine_step_body,
        grid=grid,
        in_specs=[blk, dvblk, blk, blk, wblk],
        out_specs=(blk, blk, blk),
        out_shape=(nd, nd, nd),
    )

    g = prescale(x, dv2d)
    parts = prop(rowp, colp, g)
    tx1, g1, out = combine_first(x, parts, dv2d, W[0], W[1],
                                 b.reshape(1, d))
    tx0, gcur = x, g1
    for k in range(2, k_hops):
        parts = prop(rowp, colp, gcur)
        tx2, g2, out = combine_step(parts, dv2d, tx0, out, W[k])
        tx0, tx1, gcur = tx1, tx2, g2
    return out
